# Initial kernel scaffold; baseline (speedup 1.0000x reference)
import jax
import jax.numpy as jnp
from jax.experimental import pallas as pl


def kernel(positions, atomic_numbers, edge_index, mol_ids, params):
    raise NotImplementedError("write your pallas kernel here")


# speedup vs baseline: 1.5843x; 1.5843x over previous
import functools

import jax
import jax.numpy as jnp
import numpy as np
from jax import lax
from jax.experimental import pallas as pl
from jax.experimental.pallas import tpu as pltpu
from jax.experimental.pallas import tpu_sc as plsc

N = 10000
E = 160000
F = 128
HF = 64
NRBF = 50
CUTOFF = 5.0
NI = 3
ZMAX = 100
NMOL = 16
LIMIT = 1.0
EPS = 1e-8

NPAD = 10240
EPAD = 163840
PW = 16
EB = 640
NB = 1024
ET = EPAD // 16
CF = 128
NF = ET // CF
EW = EPAD // 32
CCH = 128
NCH = EW // CCH
RPS = NPAD // 16
COEFF = -0.5 / (CUTOFF / NRBF) ** 2
PI = float(np.pi)

_INTERPRET = False


def _ssp(x):
    return jnp.maximum(x, 0.0) + jnp.log1p(jnp.exp(-jnp.abs(x))) - np.log(2.0)


def _sig(x):
    e = jnp.exp(-jnp.abs(x))
    return jnp.where(x >= 0, 1.0 / (1.0 + e), e / (1.0 + e))


def _dot(a, b):
    return jnp.dot(a, b, preferred_element_type=jnp.float32)


def _half(full, h):
    return jnp.where(h == 0, full[:, :HF], full[:, HF:])


def _split(w):
    return jnp.stack([w[:, :HF], w[:, HF:]])


def _embed_body(z_ref, emb_ref, win_ref, x0_ref, xw2_ref):
    z = z_ref[0, 0, :]
    oh = (z[:, None] == lax.broadcasted_iota(jnp.int32, (NB, ZMAX), 1)).astype(jnp.float32)
    x0 = _dot(oh, emb_ref[...])
    rows = pl.program_id(0) * NB + lax.broadcasted_iota(jnp.int32, (NB, 1), 0)
    x0 = jnp.where(rows < N, x0, 0.0)
    x0_ref[...] = x0
    h = pl.program_id(1)
    xw2_ref[0] = _dot(x0, win_ref[0])


def _tc_embed(z3d, emb, win2):
    return pl.pallas_call(
        _embed_body,
        grid=(NPAD // NB, 2),
        in_specs=[
            pl.BlockSpec((1, 1, NB), lambda i, h: (i, 0, 0)),
            pl.BlockSpec((ZMAX, F), lambda i, h: (0, 0)),
            pl.BlockSpec((1, F, HF), lambda i, h: (h, 0, 0)),
        ],
        out_specs=[pl.BlockSpec((NB, F), lambda i, h: (i, 0)),
                   pl.BlockSpec((1, NB, HF), lambda i, h: (h, i, 0))],
        out_shape=[jax.ShapeDtypeStruct((NPAD, F), jnp.float32),
                   jax.ShapeDtypeStruct((2, NPAD, HF), jnp.float32)],
        interpret=_INTERPRET,
    )(z3d, emb, win2)


def _centers_row():
    return (lax.broadcasted_iota(jnp.int32, (1, NRBF), 1).astype(jnp.float32)
            * (CUTOFF / (NRBF - 1)))


def _edge_geom(pj, pi):
    rij = pj - pi
    d = jnp.sqrt(jnp.sum(rij * rij, axis=1, keepdims=True) + 1e-12)
    centers = _centers_row()
    delta = d - centers
    rbf = jnp.exp(COEFF * delta * delta)
    inside = (d < CUTOFF).astype(jnp.float32)
    fcut = 0.5 * (jnp.cos(d * (PI / CUTOFF)) + 1.0) * inside
    return rij, d, rbf, fcut, inside


def _edge_mlp_body(pj_ref, pi_ref, wf1_ref, bf1_ref, wf2_ref, bf2_ref,
                   w0_ref, w1_ref, w2_ref):
    _, _, rbf, fcut, _ = _edge_geom(pj_ref[...], pi_ref[...])
    outs = (w0_ref, w1_ref, w2_ref)
    for l in range(NI):
        h1 = _dot(rbf, wf1_ref[l]) + bf1_ref[l]
        a1 = _ssp(h1)
        wraw = _dot(a1, wf2_ref[l, 0]) + bf2_ref[l, 0]
        outs[l][0] = wraw * fcut


def _tc_edge_mlp(pj, pi, wf1s, bf1s, wf2s2, bf2s2):
    return pl.pallas_call(
        _edge_mlp_body,
        grid=(EPAD // EB, 2),
        in_specs=[
            pl.BlockSpec((EB, PW), lambda i, h: (i, 0)),
            pl.BlockSpec((EB, PW), lambda i, h: (i, 0)),
            pl.BlockSpec((NI, NRBF, F), lambda i, h: (0, 0, 0)),
            pl.BlockSpec((NI, 1, F), lambda i, h: (0, 0, 0)),
            pl.BlockSpec((NI, 1, F, HF), lambda i, h: (0, h, 0, 0)),
            pl.BlockSpec((NI, 1, 1, HF), lambda i, h: (0, h, 0, 0)),
        ],
        out_specs=[pl.BlockSpec((1, EB, HF), lambda i, h: (h, i, 0))] * NI,
        out_shape=[jax.ShapeDtypeStruct((2, EPAD, HF), jnp.float32)] * NI,
        interpret=_INTERPRET,
    )(pj, pi, wf1s, bf1s, wf2s2, bf2s2)


def _node_body(agglo_ref, agghi_ref, x_ref, w1_ref, b1_ref, w2_ref, b2_ref,
               winn_ref, agg_ref, xn_ref, xw2_ref):
    agg = jnp.concatenate([agglo_ref[0], agghi_ref[0]], axis=1)
    agg_ref[...] = agg
    t = _dot(agg, w1_ref[...]) + b1_ref[...]
    v = _dot(_ssp(t), w2_ref[...]) + b2_ref[...]
    xn = x_ref[...] + v
    xn_ref[...] = xn
    xw2_ref[0] = _dot(xn, winn_ref[0])


def _tc_node(agg2, x, w1, b1, w2, b2, winn2):
    return pl.pallas_call(
        _node_body,
        grid=(NPAD // NB, 2),
        in_specs=[
            pl.BlockSpec((1, NB, HF), lambda i, h: (0, i, 0)),
            pl.BlockSpec((1, NB, HF), lambda i, h: (1, i, 0)),
            pl.BlockSpec((NB, F), lambda i, h: (i, 0)),
            pl.BlockSpec((F, F), lambda i, h: (0, 0)),
            pl.BlockSpec((1, F), lambda i, h: (0, 0)),
            pl.BlockSpec((F, F), lambda i, h: (0, 0)),
            pl.BlockSpec((1, F), lambda i, h: (0, 0)),
            pl.BlockSpec((1, F, HF), lambda i, h: (h, 0, 0)),
        ],
        out_specs=[pl.BlockSpec((NB, F), lambda i, h: (i, 0)),
                   pl.BlockSpec((NB, F), lambda i, h: (i, 0)),
                   pl.BlockSpec((1, NB, HF), lambda i, h: (h, i, 0))],
        out_shape=[jax.ShapeDtypeStruct((NPAD, F), jnp.float32),
                   jax.ShapeDtypeStruct((NPAD, F), jnp.float32),
                   jax.ShapeDtypeStruct((2, NPAD, HF), jnp.float32)],
        interpret=_INTERPRET,
    )(agg2, agg2, x, w1, b1, w2, b2, winn2)


def _node_last_body(agglo_ref, agghi_ref, x_ref, w1_ref, b1_ref, w2_ref,
                    b2_ref, wa1_ref, ba1_ref, wa2_ref, ba2_ref, wa1t_ref,
                    mol_ref, agg_ref, emol_ref, dx_ref):
    agg = jnp.concatenate([agglo_ref[0], agghi_ref[0]], axis=1)
    agg_ref[...] = agg
    t = _dot(agg, w1_ref[...]) + b1_ref[...]
    v = _dot(_ssp(t), w2_ref[...]) + b2_ref[...]
    x3 = x_ref[...] + v
    y1 = _dot(x3, wa1_ref[...]) + ba1_ref[...]
    z = _ssp(y1)
    wa2 = wa2_ref[...]
    e_atom = jnp.sum(z * wa2, axis=1, keepdims=True) + ba2_ref[...]
    mol = mol_ref[0, 0, :]
    oh = (mol[:, None] == lax.broadcasted_iota(jnp.int32, (NB, 128), 1))
    part = jnp.sum(jnp.where(oh, e_atom, 0.0), axis=0, keepdims=True)

    @pl.when(pl.program_id(0) == 0)
    def _():
        emol_ref[...] = jnp.zeros_like(emol_ref)

    emol_ref[...] += part
    dy1 = wa2 * _sig(y1)
    dx_ref[...] = _dot(dy1, wa1t_ref[...])


def _tc_node_last(agg2, x, w1, b1, w2, b2, wa1, ba1, wa2r, ba2, wa1t, mol3d):
    return pl.pallas_call(
        _node_last_body,
        grid=(NPAD // NB,),
        in_specs=[
            pl.BlockSpec((1, NB, HF), lambda i: (0, i, 0)),
            pl.BlockSpec((1, NB, HF), lambda i: (1, i, 0)),
            pl.BlockSpec((NB, F), lambda i: (i, 0)),
            pl.BlockSpec((F, F), lambda i: (0, 0)),
            pl.BlockSpec((1, F), lambda i: (0, 0)),
            pl.BlockSpec((F, F), lambda i: (0, 0)),
            pl.BlockSpec((1, F), lambda i: (0, 0)),
            pl.BlockSpec((F, F // 2), lambda i: (0, 0)),
            pl.BlockSpec((1, F // 2), lambda i: (0, 0)),
            pl.BlockSpec((1, F // 2), lambda i: (0, 0)),
            pl.BlockSpec((1, 1), lambda i: (0, 0)),
            pl.BlockSpec((F // 2, F), lambda i: (0, 0)),
            pl.BlockSpec((1, 1, NB), lambda i: (i, 0, 0)),
        ],
        out_specs=[pl.BlockSpec((NB, F), lambda i: (i, 0)),
                   pl.BlockSpec((1, 128), lambda i: (0, 0)),
                   pl.BlockSpec((NB, F), lambda i: (i, 0))],
        out_shape=[jax.ShapeDtypeStruct((NPAD, F), jnp.float32),
                   jax.ShapeDtypeStruct((1, 128), jnp.float32),
                   jax.ShapeDtypeStruct((NPAD, F), jnp.float32)],
        interpret=_INTERPRET,
    )(agg2, agg2, x, w1, b1, w2, b2, wa1, ba1, wa2r, ba2, wa1t, mol3d)


def _bwd_node_first_body(dx_ref, agg_ref, w1_ref, b1_ref, w2t_ref, w1t_ref,
                         dagg2_ref):
    t = _dot(agg_ref[...], w1_ref[...]) + b1_ref[...]
    du = _dot(dx_ref[...], w2t_ref[...])
    dagg2_ref[0] = _dot(du * _sig(t), w1t_ref[0])


def _tc_bwd_node_first(dx, agg, w1, b1, w2t, w1t2):
    return pl.pallas_call(
        _bwd_node_first_body,
        grid=(NPAD // NB, 2),
        in_specs=[
            pl.BlockSpec((NB, F), lambda i, h: (i, 0)),
            pl.BlockSpec((NB, F), lambda i, h: (i, 0)),
            pl.BlockSpec((F, F), lambda i, h: (0, 0)),
            pl.BlockSpec((1, F), lambda i, h: (0, 0)),
            pl.BlockSpec((F, F), lambda i, h: (0, 0)),
            pl.BlockSpec((1, F, HF), lambda i, h: (h, 0, 0)),
        ],
        out_specs=[pl.BlockSpec((1, NB, HF), lambda i, h: (h, i, 0))],
        out_shape=[jax.ShapeDtypeStruct((2, NPAD, HF), jnp.float32)],
        interpret=_INTERPRET,
    )(dx, agg, w1, b1, w2t, w1t2)[0]


def _bwd_node_body(dxp_ref, dxwlo_ref, dxwhi_ref, wint_ref, agg_ref, w1_ref,
                   b1_ref, w2t_ref, w1t_ref, dx_ref, dagg2_ref):
    dxw = jnp.concatenate([dxwlo_ref[0], dxwhi_ref[0]], axis=1)
    dx = dxp_ref[...] + _dot(dxw, wint_ref[...])
    dx_ref[...] = dx
    t = _dot(agg_ref[...], w1_ref[...]) + b1_ref[...]
    du = _dot(dx, w2t_ref[...])
    dagg2_ref[0] = _dot(du * _sig(t), w1t_ref[0])


def _tc_bwd_node(dxp, dxw2, wint, agg, w1, b1, w2t, w1t2):
    return pl.pallas_call(
        _bwd_node_body,
        grid=(NPAD // NB, 2),
        in_specs=[
            pl.BlockSpec((NB, F), lambda i, h: (i, 0)),
            pl.BlockSpec((1, NB, HF), lambda i, h: (0, i, 0)),
            pl.BlockSpec((1, NB, HF), lambda i, h: (1, i, 0)),
            pl.BlockSpec((F, F), lambda i, h: (0, 0)),
            pl.BlockSpec((NB, F), lambda i, h: (i, 0)),
            pl.BlockSpec((F, F), lambda i, h: (0, 0)),
            pl.BlockSpec((1, F), lambda i, h: (0, 0)),
            pl.BlockSpec((F, F), lambda i, h: (0, 0)),
            pl.BlockSpec((1, F, HF), lambda i, h: (h, 0, 0)),
        ],
        out_specs=[pl.BlockSpec((NB, F), lambda i, h: (i, 0)),
                   pl.BlockSpec((1, NB, HF), lambda i, h: (h, i, 0))],
        out_shape=[jax.ShapeDtypeStruct((NPAD, F), jnp.float32),
                   jax.ShapeDtypeStruct((2, NPAD, HF), jnp.float32)],
        interpret=_INTERPRET,
    )(dxp, dxw2, dxw2, wint, agg, w1, b1, w2t, w1t2)


def _edge_bwd_body(pj_ref, pi_ref, d0lo_ref, d0hi_ref, d1lo_ref, d1hi_ref,
                   d2lo_ref, d2hi_ref, wf1_ref, bf1_ref, wf2_ref, bf2_ref,
                   wf2t_ref, wf1t_ref, vec_ref, nvec_ref):
    rij, d, rbf, fcut, inside = _edge_geom(pj_ref[...], pi_ref[...])
    centers = _centers_row()
    dfcut_dd = (-0.5 * PI / CUTOFF) * jnp.sin(d * (PI / CUTOFF)) * inside
    drbf_dd = rbf * (2.0 * COEFF) * (d - centers)
    dws = ((d0lo_ref, d0hi_ref), (d1lo_ref, d1hi_ref), (d2lo_ref, d2hi_ref))
    dd = jnp.zeros_like(d)
    for l in range(NI):
        h1 = _dot(rbf, wf1_ref[l]) + bf1_ref[l]
        a1 = _ssp(h1)
        wraw = _dot(a1, wf2_ref[l]) + bf2_ref[l]
        dwij = jnp.concatenate([dws[l][0][0], dws[l][1][0]], axis=1)
        dwraw = dwij * fcut
        dfcut = jnp.sum(dwij * wraw, axis=1, keepdims=True)
        da1 = _dot(dwraw, wf2t_ref[l])
        dh1 = da1 * _sig(h1)
        drbf = _dot(dh1, wf1t_ref[l])
        dd += jnp.sum(drbf * drbf_dd, axis=1, keepdims=True) + dfcut * dfcut_dd
    vec = (dd / d) * rij
    vec_ref[...] = vec
    nvec_ref[...] = -vec


def _tc_edge_bwd(pj, pi, dw0, dw1, dw2, wf1s, bf1s, wf2s, bf2s, wf2ts, wf1ts):
    espec = pl.BlockSpec((EB, PW), lambda i: (i, 0))
    lo = pl.BlockSpec((1, EB, HF), lambda i: (0, i, 0))
    hi = pl.BlockSpec((1, EB, HF), lambda i: (1, i, 0))
    return pl.pallas_call(
        _edge_bwd_body,
        grid=(EPAD // EB,),
        in_specs=[
            espec, espec, lo, hi, lo, hi, lo, hi,
            pl.BlockSpec((NI, NRBF, F), lambda i: (0, 0, 0)),
            pl.BlockSpec((NI, 1, F), lambda i: (0, 0, 0)),
            pl.BlockSpec((NI, F, F), lambda i: (0, 0, 0)),
            pl.BlockSpec((NI, 1, F), lambda i: (0, 0, 0)),
            pl.BlockSpec((NI, F, F), lambda i: (0, 0, 0)),
            pl.BlockSpec((NI, F, NRBF), lambda i: (0, 0, 0)),
        ],
        out_specs=[espec] * 2,
        out_shape=[jax.ShapeDtypeStruct((EPAD, PW), jnp.float32)] * 2,
        interpret=_INTERPRET,
    )(pj, pi, dw0, dw0, dw1, dw1, dw2, dw2,
      wf1s, bf1s, wf2s, bf2s, wf2ts, wf1ts)


def _norms_body(dposp_ref, mol_ref, f_ref, mm_ref):
    f = -(dposp_ref[0] + dposp_ref[1])
    f_ref[...] = f
    nrm = jnp.sqrt(jnp.sum(f * f, axis=1, keepdims=True))
    mol = mol_ref[0, 0, :]
    oh = (mol[:, None] == lax.broadcasted_iota(jnp.int32, (NB, 128), 1))
    masked = jnp.where(oh, nrm, -1.0)
    part = jnp.max(masked, axis=0, keepdims=True)

    @pl.when(pl.program_id(0) == 0)
    def _():
        mm_ref[...] = jnp.full_like(mm_ref, -1.0)

    mm_ref[...] = jnp.maximum(mm_ref[...], part)


def _tc_norms(dposp, mol3d):
    return pl.pallas_call(
        _norms_body,
        grid=(NPAD // NB,),
        in_specs=[
            pl.BlockSpec((2, NB, PW), lambda i: (0, i, 0)),
            pl.BlockSpec((1, 1, NB), lambda i: (i, 0, 0)),
        ],
        out_specs=[pl.BlockSpec((NB, PW), lambda i: (i, 0)),
                   pl.BlockSpec((1, 128), lambda i: (0, 0))],
        out_shape=[jax.ShapeDtypeStruct((NPAD, PW), jnp.float32),
                   jax.ShapeDtypeStruct((1, 128), jnp.float32)],
        interpret=_INTERPRET,
    )(dposp, mol3d)


def _action_body(f_ref, mm_ref, mol_ref, act_ref):
    mm = jnp.maximum(mm_ref[...], EPS)
    coef = jnp.minimum(LIMIT / mm, 1.0)
    mol = mol_ref[0, 0, :]
    oh = (mol[:, None] == lax.broadcasted_iota(jnp.int32, (NB, 128), 1))
    catom = jnp.sum(jnp.where(oh, coef, 0.0), axis=1, keepdims=True)
    act_ref[...] = f_ref[...] * catom


def _tc_action(forces, mm, mol3d):
    return pl.pallas_call(
        _action_body,
        grid=(NPAD // NB,),
        in_specs=[
            pl.BlockSpec((NB, PW), lambda i: (i, 0)),
            pl.BlockSpec((1, 128), lambda i: (0, 0)),
            pl.BlockSpec((1, 1, NB), lambda i: (i, 0, 0)),
        ],
        out_specs=[pl.BlockSpec((NB, PW), lambda i: (i, 0))],
        out_shape=[jax.ShapeDtypeStruct((NPAD, PW), jnp.float32)],
        interpret=_INTERPRET,
    )(forces, mm, mol3d)[0]


def _sc_mesh():
    return plsc.VectorSubcoreMesh(core_axis_name="c", subcore_axis_name="s")


def _zero_vmem(buf, rows, width):
    def zrow(r, _):
        for k in range(width // 16):
            buf[r, pl.ds(k * 16, 16)] = jnp.zeros((16,), jnp.float32)
        return 0
    lax.fori_loop(0, rows, zrow, 0)


def _zero_shared(buf, shared, s, rows):
    for k in range(RPS // rows):
        pltpu.sync_copy(buf, shared.at[pl.ds(s * RPS + k * rows, rows)])


def _vcopy(dst, src, n, off=None):
    for k in range(n // 16):
        sl = pl.ds(k * 16, 16)
        v = src[sl]
        dst[sl] = v if off is None else v + off


def sc_gather(table, idx):
    @functools.partial(
        pl.kernel,
        out_type=jax.ShapeDtypeStruct((EPAD, PW), jnp.float32),
        mesh=_sc_mesh(),
        compiler_params=pltpu.CompilerParams(use_tc_tiling_on_sc=False),
        scratch_types=[
            pltpu.VMEM((CCH,), jnp.int32),
            pltpu.VMEM((CCH,), jnp.int32),
            pltpu.VMEM((CCH, PW), jnp.float32),
            pltpu.VMEM((CCH, PW), jnp.float32),
            pltpu.SemaphoreType.DMA,
            pltpu.SemaphoreType.DMA,
            pltpu.SemaphoreType.DMA,
            pltpu.SemaphoreType.DMA,
            pltpu.SemaphoreType.DMA,
            pltpu.SemaphoreType.DMA,
        ],
    )
    def k(table_hbm, idx_hbm, out_hbm, i0, i1, r0, r1, si0, si1, sg0, sg1,
          so0, so1):
        wid = lax.axis_index("s") * 2 + lax.axis_index("c")
        base = wid * EW
        IV = (i0, i1)
        RV = (r0, r1)
        SI = (si0, si1)
        SG = (sg0, sg1)
        SO = (so0, so1)
        for b in range(2):
            pltpu.async_copy(idx_hbm.at[pl.ds(base + b * CCH, CCH)], IV[b], SI[b])

        def group(g, _):
            for half in range(2):
                c = 2 * g + half
                b = half
                bc = 1 - half

                @pl.when(c < NCH)
                def _():
                    pltpu.make_async_copy(idx_hbm.at[pl.ds(base, CCH)], IV[b], SI[b]).wait()

                    @pl.when(c >= 2)
                    def _():
                        pltpu.make_async_copy(RV[b], out_hbm.at[pl.ds(base, CCH)], SO[b]).wait()

                    pltpu.async_copy(table_hbm.at[IV[b]], RV[b], SG[b])

                @pl.when((c >= 1) & (c - 1 < NCH))
                def _():
                    cc = c - 1
                    pltpu.make_async_copy(table_hbm.at[IV[bc]], RV[bc], SG[bc]).wait()

                    @pl.when(cc + 2 < NCH)
                    def _():
                        pltpu.async_copy(
                            idx_hbm.at[pl.ds(base + (cc + 2) * CCH, CCH)],
                            IV[bc], SI[bc])

                    pltpu.async_copy(RV[bc], out_hbm.at[pl.ds(base + cc * CCH, CCH)], SO[bc])
            return 0

        lax.fori_loop(0, NCH // 2 + 1, group, 0)
        pltpu.make_async_copy(RV[0], out_hbm.at[pl.ds(base, CCH)], SO[0]).wait()
        pltpu.make_async_copy(RV[1], out_hbm.at[pl.ds(base, CCH)], SO[1]).wait()

    return k(table, idx)


def sc_conv_fwd(xw2, wij2, idxj, idxi):
    NBUF = 3
    L = NBUF - 1
    scr = []
    for _ in range(NBUF):
        scr += [pltpu.VMEM((CF,), jnp.int32), pltpu.VMEM((CF,), jnp.int32),
                pltpu.VMEM((CF,), jnp.int32),
                pltpu.VMEM((CF, HF), jnp.float32),
                pltpu.VMEM((CF, HF), jnp.float32)]
    scr.append(pltpu.VMEM_SHARED((NPAD, HF), jnp.float32))
    scr += [pltpu.SemaphoreType.DMA] * (5 * NBUF)

    @functools.partial(
        pl.kernel,
        out_type=jax.ShapeDtypeStruct((2 * NPAD, HF), jnp.float32),
        mesh=_sc_mesh(),
        compiler_params=pltpu.CompilerParams(use_tc_tiling_on_sc=False),
        scratch_types=scr,
    )
    def k(xw_hbm, wij_hbm, idxj_hbm, idxi_hbm, out_hbm, *s):
        slots = [s[5 * b:5 * b + 5] for b in range(NBUF)]
        agg_sh = s[5 * NBUF]
        sems = s[5 * NBUF + 1:]
        SIJ = sems[0:NBUF]
        SII = sems[NBUF:2 * NBUF]
        SG = sems[2 * NBUF:3 * NBUF]
        SW = sems[3 * NBUF:4 * NBUF]
        SS = sems[4 * NBUF:5 * NBUF]
        core = lax.axis_index("c")
        tid = lax.axis_index("s")
        base = tid * ET
        joff = core * NPAD
        woff = core * EPAD
        _zero_vmem(slots[0][3], CF, HF)
        _zero_shared(slots[0][3], agg_sh, tid, CF)
        plsc.subcore_barrier()
        for b in range(NBUF):
            off = base + b * CF
            pltpu.async_copy(idxj_hbm.at[pl.ds(off, CF)], slots[b][0], SIJ[b])
            pltpu.async_copy(idxi_hbm.at[pl.ds(off, CF)], slots[b][1], SII[b])

        def group(g, _):
            for half in range(NBUF):
                c = g * NBUF + half
                b = half
                ij, ii, isc, rows, wv = slots[b]

                @pl.when(c < NF)
                def _():
                    pltpu.make_async_copy(idxj_hbm.at[pl.ds(base, CF)], ij, SIJ[b]).wait()
                    pltpu.make_async_copy(idxi_hbm.at[pl.ds(base, CF)], ii, SII[b]).wait()

                    @pl.when(c >= NBUF)
                    def _():
                        pltpu.make_async_copy(rows, agg_sh.at[isc], SS[b]).wait()

                    _vcopy(ij, ij, CF, joff)
                    off = base + c * CF
                    pltpu.async_copy(xw_hbm.at[ij], rows, SG[b])
                    pltpu.async_copy(wij_hbm.at[pl.ds(woff + off, CF)], wv, SW[b])

                bc = (half - L) % NBUF
                ij2, ii2, isc2, rows2, wv2 = slots[bc]

                @pl.when((c >= L) & (c - L < NF))
                def _():
                    cc = c - L
                    pltpu.make_async_copy(xw_hbm.at[ij2], rows2, SG[bc]).wait()
                    pltpu.make_async_copy(wij_hbm.at[pl.ds(base, CF)], wv2, SW[bc]).wait()
                    _vcopy(isc2, ii2, CF)

                    @pl.when(cc + NBUF < NF)
                    def _():
                        off2 = base + (cc + NBUF) * CF
                        pltpu.async_copy(idxj_hbm.at[pl.ds(off2, CF)], ij2, SIJ[bc])
                        pltpu.async_copy(idxi_hbm.at[pl.ds(off2, CF)], ii2, SII[bc])

                    def mulrow(r, _2):
                        for kk in range(HF // 16):
                            sl = pl.ds(kk * 16, 16)
                            rows2[r, sl] = rows2[r, sl] * wv2[r, sl]
                        return 0

                    lax.fori_loop(0, CF, mulrow, 0)
                    pltpu.async_copy(rows2, agg_sh.at[isc2], SS[bc], add=True)
            return 0

        lax.fori_loop(0, (NF + L + NBUF - 1) // NBUF + 1, group, 0)
        for b in range(NBUF):
            pltpu.make_async_copy(slots[b][3], agg_sh.at[slots[b][2]], SS[b]).wait()
        plsc.subcore_barrier()
        for k8 in range(RPS // CF):
            r0 = tid * RPS + k8 * CF
            pltpu.sync_copy(agg_sh.at[pl.ds(r0, CF)],
                            out_hbm.at[pl.ds(core * NPAD + r0, CF)])

    return k(xw2, wij2, idxj, idxi)


def sc_conv_bwd(dagg2, xw2, wij2, idxj, idxi):
    NBUF = 2
    L = NBUF - 1
    scr = []
    for _ in range(NBUF):
        scr += [pltpu.VMEM((CF,), jnp.int32), pltpu.VMEM((CF,), jnp.int32),
                pltpu.VMEM((CF,), jnp.int32),
                pltpu.VMEM((CF, HF), jnp.float32),
                pltpu.VMEM((CF, HF), jnp.float32),
                pltpu.VMEM((CF, HF), jnp.float32)]
    scr.append(pltpu.VMEM_SHARED((NPAD, HF), jnp.float32))
    scr += [pltpu.SemaphoreType.DMA] * (7 * NBUF)

    @functools.partial(
        pl.kernel,
        out_type=(jax.ShapeDtypeStruct((2 * NPAD, HF), jnp.float32),
                  jax.ShapeDtypeStruct((2 * EPAD, HF), jnp.float32)),
        mesh=_sc_mesh(),
        compiler_params=pltpu.CompilerParams(use_tc_tiling_on_sc=False),
        scratch_types=scr,
    )
    def k(dagg_hbm, xw_hbm, wij_hbm, idxj_hbm, idxi_hbm, dxw_hbm, dwij_hbm, *s):
        slots = [s[6 * b:6 * b + 6] for b in range(NBUF)]
        dxw_sh = s[6 * NBUF]
        sems = s[6 * NBUF + 1:]
        SIJ = sems[0:NBUF]
        SII = sems[NBUF:2 * NBUF]
        SG1 = sems[2 * NBUF:3 * NBUF]
        SG2 = sems[3 * NBUF:4 * NBUF]
        SW = sems[4 * NBUF:5 * NBUF]
        SS = sems[5 * NBUF:6 * NBUF]
        SD = sems[6 * NBUF:7 * NBUF]
        core = lax.axis_index("c")
        tid = lax.axis_index("s")
        base = tid * ET
        joff = core * NPAD
        woff = core * EPAD
        _zero_vmem(slots[0][3], CF, HF)
        _zero_shared(slots[0][3], dxw_sh, tid, CF)
        plsc.subcore_barrier()
        for b in range(NBUF):
            off = base + b * CF
            pltpu.async_copy(idxj_hbm.at[pl.ds(off, CF)], slots[b][0], SIJ[b])
            pltpu.async_copy(idxi_hbm.at[pl.ds(off, CF)], slots[b][1], SII[b])

        def group(g, _):
            for half in range(NBUF):
                c = g * NBUF + half
                b = half
                ij, ii, isc, dm, xwj, wv = slots[b]

                @pl.when(c < NF)
                def _():
                    pltpu.make_async_copy(idxj_hbm.at[pl.ds(base, CF)], ij, SIJ[b]).wait()
                    pltpu.make_async_copy(idxi_hbm.at[pl.ds(base, CF)], ii, SII[b]).wait()

                    @pl.when(c >= NBUF)
                    def _():
                        pltpu.make_async_copy(wv, dxw_sh.at[isc], SS[b]).wait()
                        pltpu.make_async_copy(xwj, dwij_hbm.at[pl.ds(base, CF)], SD[b]).wait()

                    _vcopy(ij, ij, CF, joff)
                    _vcopy(ii, ii, CF, joff)
                    off = base + c * CF
                    pltpu.async_copy(dagg_hbm.at[ii], dm, SG1[b])
                    pltpu.async_copy(xw_hbm.at[ij], xwj, SG2[b])
                    pltpu.async_copy(wij_hbm.at[pl.ds(woff + off, CF)], wv, SW[b])

                bc = (half - L) % NBUF
                ij2, ii2, isc2, dm2, xwj2, wv2 = slots[bc]

                @pl.when((c >= L) & (c - L < NF))
                def _():
                    cc = c - L
                    pltpu.make_async_copy(dagg_hbm.at[ii2], dm2, SG1[bc]).wait()
                    pltpu.make_async_copy(xw_hbm.at[ij2], xwj2, SG2[bc]).wait()
                    pltpu.make_async_copy(wij_hbm.at[pl.ds(base, CF)], wv2, SW[bc]).wait()
                    _vcopy(isc2, ij2, CF, -joff)

                    @pl.when(cc + NBUF < NF)
                    def _():
                        off2 = base + (cc + NBUF) * CF
                        pltpu.async_copy(idxj_hbm.at[pl.ds(off2, CF)], ij2, SIJ[bc])
                        pltpu.async_copy(idxi_hbm.at[pl.ds(off2, CF)], ii2, SII[bc])

                    def mulrow(r, _2):
                        for kk in range(HF // 16):
                            sl = pl.ds(kk * 16, 16)
                            a = dm2[r, sl]
                            xwj2[r, sl] = a * xwj2[r, sl]
                            wv2[r, sl] = a * wv2[r, sl]
                        return 0

                    lax.fori_loop(0, CF, mulrow, 0)
                    off3 = base + cc * CF
                    pltpu.async_copy(xwj2, dwij_hbm.at[pl.ds(woff + off3, CF)], SD[bc])
                    pltpu.async_copy(wv2, dxw_sh.at[isc2], SS[bc], add=True)
            return 0

        lax.fori_loop(0, (NF + L + NBUF - 1) // NBUF + 1, group, 0)
        for b in range(NBUF):
            pltpu.make_async_copy(slots[b][5], dxw_sh.at[slots[b][2]], SS[b]).wait()
            pltpu.make_async_copy(slots[b][4], dwij_hbm.at[pl.ds(base, CF)], SD[b]).wait()
        plsc.subcore_barrier()
        for k8 in range(RPS // CF):
            r0 = tid * RPS + k8 * CF
            pltpu.sync_copy(dxw_sh.at[pl.ds(r0, CF)],
                            dxw_hbm.at[pl.ds(core * NPAD + r0, CF)])

    return k(dagg2, xw2, wij2, idxj, idxi)


def sc_scatter_vec(vec, nvec, idxj, idxi):
    @functools.partial(
        pl.kernel,
        out_type=jax.ShapeDtypeStruct((2 * NPAD, PW), jnp.float32),
        mesh=_sc_mesh(),
        compiler_params=pltpu.CompilerParams(use_tc_tiling_on_sc=False),
        scratch_types=[
            pltpu.VMEM((CCH,), jnp.int32),
            pltpu.VMEM((CCH,), jnp.int32),
            pltpu.VMEM((CCH, PW), jnp.float32),
            pltpu.VMEM((CCH, PW), jnp.float32),
            pltpu.VMEM_SHARED((NPAD, PW), jnp.float32),
        ],
    )
    def k(vec_hbm, nvec_hbm, idxj_hbm, idxi_hbm, out_hbm,
          idxj_v, idxi_v, v_v, nv_v, dpos_sh):
        c = lax.axis_index("c")
        s = lax.axis_index("s")
        wid = s * 2 + c
        _zero_vmem(v_v, CCH, PW)
        _zero_shared(v_v, dpos_sh, s, CCH)
        plsc.subcore_barrier()

        def body(ch, _):
            off = wid * EW + ch * CCH
            pltpu.sync_copy(idxj_hbm.at[pl.ds(off, CCH)], idxj_v)
            pltpu.sync_copy(idxi_hbm.at[pl.ds(off, CCH)], idxi_v)
            pltpu.sync_copy(vec_hbm.at[pl.ds(off, CCH)], v_v)
            pltpu.sync_copy(nvec_hbm.at[pl.ds(off, CCH)], nv_v)
            pltpu.sync_copy(v_v, dpos_sh.at[idxj_v], add=True)
            pltpu.sync_copy(nv_v, dpos_sh.at[idxi_v], add=True)
            return 0

        lax.fori_loop(0, NCH, body, 0)
        plsc.subcore_barrier()
        for k8 in range(RPS // CCH):
            r0 = s * RPS + k8 * CCH
            pltpu.sync_copy(dpos_sh.at[pl.ds(r0, CCH)],
                            out_hbm.at[pl.ds(c * NPAD + r0, CCH)])

    return k(vec, nvec, idxj, idxi)


def kernel(positions, atomic_numbers, edge_index, mol_ids, params):
    f32 = jnp.float32
    pos_pad = jnp.zeros((NPAD, PW), f32).at[:N, :3].set(positions)
    z3d = (jnp.zeros((NPAD,), jnp.int32).at[:N].set(atomic_numbers)
           .reshape(NPAD // NB, 1, NB))
    idxi = jnp.full((EPAD,), N, jnp.int32).at[:E].set(edge_index[0])
    idxj = jnp.full((EPAD,), N, jnp.int32).at[:E].set(edge_index[1])
    mol3d = (jnp.full((NPAD,), NMOL, jnp.int32).at[:N].set(mol_ids)
             .reshape(NPAD // NB, 1, NB))

    ly = params['interactions']
    wf1s = jnp.stack([l['Wf1'] for l in ly])
    bf1s = jnp.stack([l['bf1'][None, :] for l in ly])
    wf2s = jnp.stack([l['Wf2'] for l in ly])
    bf2s = jnp.stack([l['bf2'][None, :] for l in ly])
    wf2s2 = jnp.stack([_split(l['Wf2']) for l in ly])
    bf2s2 = jnp.stack([jnp.stack([l['bf2'][None, :HF], l['bf2'][None, HF:]])
                       for l in ly])
    wf2ts = jnp.stack([l['Wf2'].T for l in ly])
    wf1ts = jnp.stack([l['Wf1'].T for l in ly])

    x0, xw0 = _tc_embed(z3d, params['embedding'], _split(ly[0]['W_in']))
    pj = sc_gather(pos_pad, idxj)
    pi = sc_gather(pos_pad, idxi)
    wij0, wij1, wij2 = _tc_edge_mlp(pj, pi, wf1s, bf1s, wf2s2, bf2s2)
    wijs = (wij0, wij1, wij2)

    xs, xws, aggs = [x0], [xw0], []
    for l in range(NI - 1):
        agg2 = sc_conv_fwd(xws[l].reshape(2 * NPAD, HF),
                           wijs[l].reshape(2 * EPAD, HF),
                           idxj, idxi).reshape(2, NPAD, HF)
        agg, xn, xwn = _tc_node(agg2, xs[l], ly[l]['W1'], ly[l]['b1'][None, :],
                                ly[l]['W2'], ly[l]['b2'][None, :],
                                _split(ly[l + 1]['W_in']))
        aggs.append(agg)
        xs.append(xn)
        xws.append(xwn)

    agg2 = sc_conv_fwd(xws[2].reshape(2 * NPAD, HF),
                       wijs[2].reshape(2 * EPAD, HF),
                       idxj, idxi).reshape(2, NPAD, HF)
    aggL, emol128, dx3 = _tc_node_last(
        agg2, xs[2], ly[2]['W1'], ly[2]['b1'][None, :], ly[2]['W2'],
        ly[2]['b2'][None, :], params['Wa1'], params['ba1'][None, :],
        params['Wa2'][:, 0][None, :], params['ba2'][None, :],
        params['Wa1'].T, mol3d)
    aggs.append(aggL)

    dagg2_2 = _tc_bwd_node_first(dx3, aggs[2], ly[2]['W1'],
                                 ly[2]['b1'][None, :], ly[2]['W2'].T,
                                 _split(ly[2]['W1'].T))
    dxw2, dwij2 = sc_conv_bwd(dagg2_2.reshape(2 * NPAD, HF),
                              xws[2].reshape(2 * NPAD, HF),
                              wijs[2].reshape(2 * EPAD, HF), idxj, idxi)
    dx2, dagg1_2 = _tc_bwd_node(dx3, dxw2.reshape(2, NPAD, HF),
                                ly[2]['W_in'].T, aggs[1], ly[1]['W1'],
                                ly[1]['b1'][None, :], ly[1]['W2'].T,
                                _split(ly[1]['W1'].T))
    dxw1, dwij1 = sc_conv_bwd(dagg1_2.reshape(2 * NPAD, HF),
                              xws[1].reshape(2 * NPAD, HF),
                              wijs[1].reshape(2 * EPAD, HF), idxj, idxi)
    _, dagg0_2 = _tc_bwd_node(dx2, dxw1.reshape(2, NPAD, HF),
                              ly[1]['W_in'].T, aggs[0], ly[0]['W1'],
                              ly[0]['b1'][None, :], ly[0]['W2'].T,
                              _split(ly[0]['W1'].T))
    _, dwij0 = sc_conv_bwd(dagg0_2.reshape(2 * NPAD, HF),
                           xws[0].reshape(2 * NPAD, HF),
                           wijs[0].reshape(2 * EPAD, HF), idxj, idxi)

    vec, nvec = _tc_edge_bwd(pj, pi, dwij0.reshape(2, EPAD, HF),
                             dwij1.reshape(2, EPAD, HF),
                             dwij2.reshape(2, EPAD, HF),
                             wf1s, bf1s, wf2s, bf2s, wf2ts, wf1ts)
    dposp = sc_scatter_vec(vec, nvec, idxj, idxi).reshape(2, NPAD, PW)
    forces, mm = _tc_norms(dposp, mol3d)
    act_pad = _tc_action(forces, mm, mol3d)

    return (act_pad[:N, :3], emol128[0, :NMOL])
